# baseline (speedup 1.0000x reference)
import functools

import jax
import jax.numpy as jnp
from jax import lax
from jax.experimental import pallas as pl
from jax.experimental.pallas import tpu as pltpu
from jax.experimental.pallas import tpu_sc as plsc

_VOCAB = 8
_DIM = 16
_OUT = 8
_B = 16384
_SEQ = 200
_N = _B * _SEQ
_NW = 32
_PER_W = _N // _NW
_K = 2
_CHUNK = _K * 1024
_NCH = _PER_W // _CHUNK
_CPL = 128 // _K
_L = 16
_LSTRIDE = _B * _OUT

_mesh = plsc.VectorSubcoreMesh(core_axis_name="c", subcore_axis_name="s")


@functools.partial(
    pl.kernel,
    out_type=jax.ShapeDtypeStruct((_N * _OUT,), jnp.float32),
    mesh=_mesh,
    compiler_params=pltpu.CompilerParams(
        needs_layout_passes=False, use_tc_tiling_on_sc=False),
    scratch_types=[
        pltpu.VMEM((_VOCAB, _DIM), jnp.float32),
        pltpu.VMEM((_VOCAB, _DIM), jnp.float32),
        pltpu.VMEM((_L,), jnp.float32),
        pltpu.VMEM((_OUT, _VOCAB), jnp.float32),
        pltpu.VMEM((_CHUNK,), jnp.int32),
        pltpu.VMEM((_CHUNK,), jnp.int32),
        pltpu.VMEM((_CHUNK * _OUT,), jnp.float32),
        pltpu.VMEM((_CHUNK * _OUT,), jnp.float32),
        pltpu.SemaphoreType.DMA,
        pltpu.SemaphoreType.DMA,
        pltpu.SemaphoreType.DMA,
        pltpu.SemaphoreType.DMA,
    ],
)
def _sc_lookup(emb_hbm, fcw_hbm, fcb_hbm, xt_hbm, out_hbm,
               emb_v, fcw_v, fcb_v, tab_v,
               idx0, idx1, rows0, rows1, sin0, sin1, sout0, sout1):
    idx = (idx0, idx1)
    rows = (rows0, rows1)
    sin = (sin0, sin1)
    sout = (sout0, sout1)
    wid = lax.axis_index("s") * 2 + lax.axis_index("c")
    base_w = wid * _NCH

    def in_src(g):
        return xt_hbm.at[pl.ds((base_w + g) * _CHUNK, _CHUNK)]

    for b in range(2):
        pltpu.async_copy(in_src(b), idx[b], sin[b])

    pltpu.sync_copy(emb_hbm, emb_v)
    pltpu.sync_copy(fcw_hbm, fcw_v)
    pltpu.sync_copy(fcb_hbm, fcb_v)

    iota = lax.iota(jnp.int32, _L)

    def splat(val):
        return jnp.broadcast_to(jnp.int32(val), (_L,))

    for j in range(4):
        p = iota + splat(j * _L)
        v_idx = lax.shift_right_logical(p, splat(3))
        k_idx = jnp.bitwise_and(p, splat(7))

        def mac_d(d, acc):
            dd = jnp.broadcast_to(d, (_L,))
            wv = plsc.load_gather(fcw_v, [v_idx, dd])
            ek = plsc.load_gather(emb_v, [k_idx, dd])
            return acc + wv * ek

        acc = lax.fori_loop(0, _DIM, mac_d, plsc.load_gather(fcb_v, [v_idx]))
        plsc.store_scatter(tab_v, [v_idx, k_idx], acc)

    def start_out(g, b):
        u = base_w + g
        l_hi = u // _CPL
        bh0 = (u % _CPL) * _K
        base = l_hi * 8 * _LSTRIDE + bh0 * 1024
        for l_lo in range(8):
            pltpu.async_copy(
                rows[b].at[pl.ds(l_lo * (_K * 1024), _K * 1024)],
                out_hbm.at[pl.ds(base + l_lo * _LSTRIDE, _K * 1024)],
                sout[b])

    def drain_out(b):
        pltpu.make_async_copy(
            rows[b], out_hbm.at[pl.ds(0, _CHUNK * _OUT)], sout[b]).wait()

    def pair_body(p, carry):
        for b in range(2):
            g = p * 2 + b
            pltpu.make_async_copy(in_src(g), idx[b], sin[b]).wait()

            @pl.when(p >= 1)
            def _wait_out():
                drain_out(b)

            @plsc.parallel_loop(0, _CHUNK, step=_L, unroll=4)
            def _tok_body(i):
                tok = idx[b][pl.ds(i, _L)]
                pos = (((i >> 7) & 7) * (_K * 1024)) | ((i >> 10) << 10) | (i & 127)
                for v in range(_OUT):
                    r = plsc.load_gather(tab_v, [splat(v), tok])
                    rows[b][pl.ds(pos + v * 128, _L)] = r

            start_out(g, b)

            @pl.when(g + 2 < _NCH)
            def _next_in():
                pltpu.async_copy(in_src(g + 2), idx[b], sin[b])
        return carry

    lax.fori_loop(0, _NCH // 2, pair_body, 0)

    for b in range(2):
        drain_out(b)


def kernel(x, embedding, fc_w, fc_b):
    x_feed = x.reshape(128, 128, 25, 8).transpose(2, 0, 3, 1).reshape(-1)
    fcb_pad = jnp.pad(fc_b, (0, _L - _VOCAB))
    flat = _sc_lookup(embedding, fc_w, fcb_pad, x_feed)
    r4 = flat.reshape(_SEQ, _B // 128, _OUT, 128)
    return r4.transpose(1, 3, 0, 2).reshape(_B, _SEQ, _OUT)


# speedup vs baseline: 1.0479x; 1.0479x over previous
import functools

import jax
import jax.numpy as jnp
from jax import lax
from jax.experimental import pallas as pl
from jax.experimental.pallas import tpu as pltpu
from jax.experimental.pallas import tpu_sc as plsc

_VOCAB = 8
_DIM = 16
_OUT = 8
_B = 16384
_SEQ = 200
_N = _B * _SEQ
_NW = 32
_PER_W = _N // _NW
_K = 4
_CHUNK = _K * 1024
_NCH = _PER_W // _CHUNK
_CPL = 128 // _K
_L = 16
_LSTRIDE = _B * _OUT

_mesh = plsc.VectorSubcoreMesh(core_axis_name="c", subcore_axis_name="s")


@functools.partial(
    pl.kernel,
    out_type=jax.ShapeDtypeStruct((_N * _OUT,), jnp.float32),
    mesh=_mesh,
    compiler_params=pltpu.CompilerParams(
        needs_layout_passes=False, use_tc_tiling_on_sc=False),
    scratch_types=[
        pltpu.VMEM((_VOCAB, _DIM), jnp.float32),
        pltpu.VMEM((_VOCAB, _DIM), jnp.float32),
        pltpu.VMEM((_L,), jnp.float32),
        pltpu.VMEM((_OUT, _VOCAB), jnp.float32),
        pltpu.VMEM((_CHUNK,), jnp.int32),
        pltpu.VMEM((_CHUNK,), jnp.int32),
        pltpu.VMEM((_CHUNK * _OUT,), jnp.float32),
        pltpu.VMEM((_CHUNK * _OUT,), jnp.float32),
        pltpu.SemaphoreType.DMA,
        pltpu.SemaphoreType.DMA,
        pltpu.SemaphoreType.DMA,
        pltpu.SemaphoreType.DMA,
    ],
)
def _sc_lookup(emb_hbm, fcw_hbm, fcb_hbm, xt_hbm, out_hbm,
               emb_v, fcw_v, fcb_v, tab_v,
               idx0, idx1, rows0, rows1, sin0, sin1, sout0, sout1):
    idx = (idx0, idx1)
    rows = (rows0, rows1)
    sin = (sin0, sin1)
    sout = (sout0, sout1)
    wid = lax.axis_index("s") * 2 + lax.axis_index("c")
    base_w = wid * _NCH

    def in_src(g):
        return xt_hbm.at[pl.ds((base_w + g) * _CHUNK, _CHUNK)]

    for b in range(2):
        pltpu.async_copy(in_src(b), idx[b], sin[b])

    pltpu.sync_copy(emb_hbm, emb_v)
    pltpu.sync_copy(fcw_hbm, fcw_v)
    pltpu.sync_copy(fcb_hbm, fcb_v)

    iota = lax.iota(jnp.int32, _L)

    def splat(val):
        return jnp.broadcast_to(jnp.int32(val), (_L,))

    for j in range(4):
        p = iota + splat(j * _L)
        v_idx = lax.shift_right_logical(p, splat(3))
        k_idx = jnp.bitwise_and(p, splat(7))

        def mac_d(d, acc):
            dd = jnp.broadcast_to(d, (_L,))
            wv = plsc.load_gather(fcw_v, [v_idx, dd])
            ek = plsc.load_gather(emb_v, [k_idx, dd])
            return acc + wv * ek

        acc = lax.fori_loop(0, _DIM, mac_d, plsc.load_gather(fcb_v, [v_idx]))
        plsc.store_scatter(tab_v, [v_idx, k_idx], acc)

    def start_out(g, b):
        u = base_w + g
        l_hi = u // _CPL
        bh0 = (u % _CPL) * _K
        base = l_hi * 8 * _LSTRIDE + bh0 * 1024
        for l_lo in range(8):
            pltpu.async_copy(
                rows[b].at[pl.ds(l_lo * (_K * 1024), _K * 1024)],
                out_hbm.at[pl.ds(base + l_lo * _LSTRIDE, _K * 1024)],
                sout[b])

    def drain_out(b):
        pltpu.make_async_copy(
            rows[b], out_hbm.at[pl.ds(0, _CHUNK * _OUT)], sout[b]).wait()

    def do_chunk(g, b, first, last):
        pltpu.make_async_copy(in_src(g), idx[b], sin[b]).wait()

        if not first:
            drain_out(b)

        @plsc.parallel_loop(0, _CHUNK, step=_L, unroll=4)
        def _tok_body(i):
            tok = idx[b][pl.ds(i, _L)]
            pos = (((i >> 7) & 7) * (_K * 1024)) | ((i >> 10) << 10) | (i & 127)
            for v in range(_OUT):
                r = plsc.load_gather(tab_v, [splat(v), tok])
                rows[b][pl.ds(pos + v * 128, _L)] = r

        start_out(g, b)

        if not last:
            @pl.when(g + 2 < _NCH)
            def _next_in():
                pltpu.async_copy(in_src(g + 2), idx[b], sin[b])

    def pair_body(p, carry):
        for b in range(2):
            g = p * 2 + b
            do_chunk(g, b, first=False, last=False)
        return carry

    do_chunk(0, 0, first=True, last=False)
    do_chunk(1, 1, first=True, last=False)
    lax.fori_loop(1, (_NCH - 1) // 2, pair_body, 0)
    do_chunk(_NCH - 1, 0, first=False, last=True)

    for b in range(2):
        drain_out(b)


def kernel(x, embedding, fc_w, fc_b):
    x_feed = x.reshape(128, 128, 25, 8).transpose(2, 0, 3, 1).reshape(-1)
    fcb_pad = jnp.pad(fc_b, (0, _L - _VOCAB))
    flat = _sc_lookup(embedding, fc_w, fcb_pad, x_feed)
    r4 = flat.reshape(_SEQ, _B // 128, _OUT, 128)
    return r4.transpose(1, 3, 0, 2).reshape(_B, _SEQ, _OUT)


# speedup vs baseline: 1.0529x; 1.0047x over previous
import functools

import jax
import jax.numpy as jnp
from jax import lax
from jax.experimental import pallas as pl
from jax.experimental.pallas import tpu as pltpu
from jax.experimental.pallas import tpu_sc as plsc

_VOCAB = 8
_DIM = 16
_OUT = 8
_B = 16384
_SEQ = 200
_N = _B * _SEQ
_NW = 32
_PER_W = _N // _NW
_K = 4
_CHUNK = _K * 1024
_NCH = _PER_W // _CHUNK
_CPL = 128 // _K
_L = 16
_LSTRIDE = _B * _OUT

_mesh = plsc.VectorSubcoreMesh(core_axis_name="c", subcore_axis_name="s")


@functools.partial(
    pl.kernel,
    out_type=jax.ShapeDtypeStruct((_N * _OUT,), jnp.float32),
    mesh=_mesh,
    compiler_params=pltpu.CompilerParams(
        needs_layout_passes=False, use_tc_tiling_on_sc=False),
    scratch_types=[
        pltpu.VMEM((_VOCAB, _DIM), jnp.float32),
        pltpu.VMEM((_VOCAB, _DIM), jnp.float32),
        pltpu.VMEM((_L,), jnp.float32),
        pltpu.VMEM((_OUT, _VOCAB), jnp.float32),
        pltpu.VMEM((_CHUNK,), jnp.int32),
        pltpu.VMEM((_CHUNK,), jnp.int32),
        pltpu.VMEM((_CHUNK * _OUT,), jnp.float32),
        pltpu.VMEM((_CHUNK * _OUT,), jnp.float32),
        pltpu.SemaphoreType.DMA,
        pltpu.SemaphoreType.DMA,
        pltpu.SemaphoreType.DMA,
        pltpu.SemaphoreType.DMA,
    ],
)
def _sc_lookup(emb_hbm, fcw_hbm, fcb_hbm, xt_hbm, out_hbm,
               emb_v, fcw_v, fcb_v, tab_v,
               idx0, idx1, rows0, rows1, sin0, sin1, sout0, sout1):
    idx = (idx0, idx1)
    rows = (rows0, rows1)
    sin = (sin0, sin1)
    sout = (sout0, sout1)
    wid = lax.axis_index("s") * 2 + lax.axis_index("c")
    base_w = wid * _NCH

    def in_src(g):
        return xt_hbm.at[pl.ds((base_w + g) * _CHUNK, _CHUNK)]

    for b in range(2):
        pltpu.async_copy(in_src(b), idx[b], sin[b])

    pltpu.sync_copy(emb_hbm, emb_v)
    pltpu.sync_copy(fcw_hbm, fcw_v)
    pltpu.sync_copy(fcb_hbm, fcb_v)

    iota = lax.iota(jnp.int32, _L)

    def splat(val):
        return jnp.broadcast_to(jnp.int32(val), (_L,))

    for j in range(4):
        p = iota + splat(j * _L)
        v_idx = lax.shift_right_logical(p, splat(3))
        k_idx = jnp.bitwise_and(p, splat(7))

        def mac_d(d, acc):
            dd = jnp.broadcast_to(d, (_L,))
            wv = plsc.load_gather(fcw_v, [v_idx, dd])
            ek = plsc.load_gather(emb_v, [k_idx, dd])
            return acc + wv * ek

        acc = lax.fori_loop(0, _DIM, mac_d, plsc.load_gather(fcb_v, [v_idx]))
        plsc.store_scatter(tab_v, [v_idx, k_idx], acc)

    def start_out(g, b):
        u = base_w + g
        l_hi = u // _CPL
        bh0 = (u % _CPL) * _K
        base = l_hi * 8 * _LSTRIDE + bh0 * 1024
        for l_lo in range(8):
            pltpu.async_copy(
                rows[b].at[pl.ds(l_lo * (_K * 1024), _K * 1024)],
                out_hbm.at[pl.ds(base + l_lo * _LSTRIDE, _K * 1024)],
                sout[b])

    def drain_out(b):
        pltpu.make_async_copy(
            rows[b], out_hbm.at[pl.ds(0, _CHUNK * _OUT)], sout[b]).wait()

    def do_chunk(g, b, first, last, unroll=4):
        pltpu.make_async_copy(in_src(g), idx[b], sin[b]).wait()

        if not first:
            drain_out(b)

        @plsc.parallel_loop(0, _CHUNK, step=_L, unroll=unroll)
        def _tok_body(i):
            tok = idx[b][pl.ds(i, _L)]
            pos = (((i >> 7) & 7) * (_K * 1024)) | ((i >> 10) << 10) | (i & 127)
            for v in range(_OUT):
                r = plsc.load_gather(tab_v, [splat(v), tok])
                rows[b][pl.ds(pos + v * 128, _L)] = r

        start_out(g, b)

        if not last:
            @pl.when(g + 2 < _NCH)
            def _next_in():
                pltpu.async_copy(in_src(g + 2), idx[b], sin[b])

    def pair_body(p, carry):
        for b in range(2):
            g = p * 2 + b
            do_chunk(g, b, first=False, last=False, unroll=8)
        return carry

    do_chunk(0, 0, first=True, last=False)
    do_chunk(1, 1, first=True, last=False)
    lax.fori_loop(1, (_NCH - 1) // 2, pair_body, 0)
    do_chunk(_NCH - 1, 0, first=False, last=True)

    for b in range(2):
        drain_out(b)


def kernel(x, embedding, fc_w, fc_b):
    x_feed = x.reshape(128, 128, 25, 8).transpose(2, 0, 3, 1).reshape(-1)
    fcb_pad = jnp.pad(fc_b, (0, _L - _VOCAB))
    flat = _sc_lookup(embedding, fc_w, fcb_pad, x_feed)
    r4 = flat.reshape(_SEQ, _B // 128, _OUT, 128)
    return r4.transpose(1, 3, 0, 2).reshape(_B, _SEQ, _OUT)
